# baseline (speedup 1.0000x reference)
import functools

import jax
import jax.numpy as jnp
from jax import lax
from jax.experimental import pallas as pl
from jax.experimental.pallas import tpu as pltpu
from jax.experimental.pallas import tpu_sc as plsc

NUM_CORES = 2
NUM_SUBCORES = 16
LANES = 16
CHUNK = 16384
NBUF = 4


def kernel(x, embed_table):
    batch, seq_len, d_model = x.shape
    n_tbl = embed_table.shape[0]
    n_workers = NUM_CORES * NUM_SUBCORES
    rows_per_w = seq_len // n_workers
    row_elems = seq_len * d_model
    n_chunks = row_elems // CHUNK

    x_flat = x.reshape(row_elems)
    tbl_flat = embed_table.reshape(n_tbl * d_model)

    mesh = plsc.VectorSubcoreMesh(
        core_axis_name="c", subcore_axis_name="s"
    )

    @functools.partial(
        pl.kernel,
        mesh=mesh,
        out_type=jax.ShapeDtypeStruct((seq_len * row_elems,), jnp.float32),
        scratch_types=[
            pltpu.VMEM((CHUNK,), jnp.float32),
            [pltpu.VMEM((CHUNK,), jnp.float32) for _ in range(NBUF)],
            [pltpu.SemaphoreType.DMA for _ in range(NBUF)],
            [pltpu.SemaphoreType.DMA for _ in range(NBUF)],
        ],
    )
    def rpe_sc(x_hbm, tbl_hbm, out_hbm, xb, tbufs, sems_in, sems_out):
        wid = lax.axis_index("s") * NUM_CORES + lax.axis_index("c")
        i0 = wid * rows_per_w

        def chunk_body(c, _):
            j_off = c * CHUNK
            pltpu.sync_copy(x_hbm.at[pl.ds(j_off, CHUNK)], xb)

            def in_copy(r):
                t_off = (seq_len - (i0 + r)) * d_model + j_off
                return pltpu.make_async_copy(
                    tbl_hbm.at[pl.ds(t_off, CHUNK)],
                    tbufs[r % NBUF],
                    sems_in[r % NBUF],
                )

            def out_copy(r):
                o_off = (i0 + r) * row_elems + j_off
                return pltpu.make_async_copy(
                    tbufs[r % NBUF],
                    out_hbm.at[pl.ds(o_off, CHUNK)],
                    sems_out[r % NBUF],
                )

            in_copy(0).start()
            in_copy(1).start()
            for r in range(rows_per_w):
                b = r % NBUF
                in_copy(r).wait()

                @plsc.parallel_loop(0, CHUNK, step=LANES, unroll=8)
                def add_body(o):
                    plsc.addupdate(
                        tbufs[b].at[pl.ds(o, LANES)],
                        xb[pl.ds(o, LANES)],
                    )
                out_copy(r).start()
                n = r + 2
                if n < rows_per_w:
                    if n >= NBUF:
                        out_copy(n - NBUF).wait()
                    in_copy(n).start()
            for r in range(rows_per_w - NBUF, rows_per_w):
                out_copy(r).wait()
            return 0

        lax.fori_loop(0, n_chunks, chunk_body, 0)

    out_flat = rpe_sc(x_flat, tbl_flat)
    return out_flat.reshape(seq_len, seq_len, d_model)


# speedup vs baseline: 1.2645x; 1.2645x over previous
import functools

import jax
import jax.numpy as jnp
from jax import lax
from jax.experimental import pallas as pl
from jax.experimental.pallas import tpu as pltpu
from jax.experimental.pallas import tpu_sc as plsc

NUM_CORES = 2
NUM_SUBCORES = 16
LANES = 16
CHUNK = 16384
NBUF = 4


def kernel(x, embed_table):
    batch, seq_len, d_model = x.shape
    n_tbl = embed_table.shape[0]
    n_workers = NUM_CORES * NUM_SUBCORES
    rows_per_w = seq_len // n_workers
    row_elems = seq_len * d_model
    n_chunks = row_elems // CHUNK

    x_flat = x.reshape(row_elems)
    tbl_flat = embed_table.reshape(n_tbl * d_model)

    mesh = plsc.VectorSubcoreMesh(
        core_axis_name="c", subcore_axis_name="s"
    )

    sup_elems = CHUNK + (rows_per_w - 1) * d_model

    @functools.partial(
        pl.kernel,
        mesh=mesh,
        out_type=jax.ShapeDtypeStruct((seq_len * row_elems,), jnp.float32),
        scratch_types=[
            pltpu.VMEM((CHUNK,), jnp.float32),
            pltpu.VMEM((sup_elems,), jnp.float32),
            [pltpu.VMEM((CHUNK,), jnp.float32) for _ in range(NBUF)],
            [pltpu.SemaphoreType.DMA for _ in range(NBUF)],
        ],
    )
    def rpe_sc(x_hbm, tbl_hbm, out_hbm, xb, sup, obufs, sems_out):
        wid = lax.axis_index("s") * NUM_CORES + lax.axis_index("c")
        i0 = wid * rows_per_w

        def chunk_body(c, _):
            j_off = c * CHUNK
            pltpu.sync_copy(x_hbm.at[pl.ds(j_off, CHUNK)], xb)
            s_off = (seq_len - (i0 + rows_per_w - 1)) * d_model + j_off
            pltpu.sync_copy(tbl_hbm.at[pl.ds(s_off, sup_elems)], sup)

            def out_copy(r):
                o_off = (i0 + r) * row_elems + j_off
                return pltpu.make_async_copy(
                    obufs[r % NBUF],
                    out_hbm.at[pl.ds(o_off, CHUNK)],
                    sems_out[r % NBUF],
                )

            for r in range(rows_per_w):
                b = r % NBUF
                if r >= NBUF:
                    out_copy(r - NBUF).wait()
                w_off = (rows_per_w - 1 - r) * d_model

                @plsc.parallel_loop(0, CHUNK, step=LANES, unroll=8)
                def add_body(o):
                    obufs[b][pl.ds(o, LANES)] = (
                        sup[pl.ds(o + w_off, LANES)]
                        + xb[pl.ds(o, LANES)]
                    )
                out_copy(r).start()
            for r in range(rows_per_w - NBUF, rows_per_w):
                out_copy(r).wait()
            return 0

        lax.fori_loop(0, n_chunks, chunk_body, 0)

    out_flat = rpe_sc(x_flat, tbl_flat)
    return out_flat.reshape(seq_len, seq_len, d_model)


# speedup vs baseline: 1.7840x; 1.4108x over previous
import functools

import jax
import jax.numpy as jnp
from jax import lax
from jax.experimental import pallas as pl
from jax.experimental.pallas import tpu as pltpu
from jax.experimental.pallas import tpu_sc as plsc

TC_ROWS = 448
ROWS_PER_BLOCK = 16

NUM_CORES = 2
NUM_SUBCORES = 16
LANES = 16
CHUNK = 16384
NBUF = 4


def _rpe_block(x_ref, tbl_ref, out_ref):
    seq_len = x_ref.shape[1]
    i0 = pl.program_id(0) * ROWS_PER_BLOCK
    xv = x_ref[0]
    base = pl.multiple_of(seq_len - i0 - ROWS_PER_BLOCK, 8)
    sup = tbl_ref[pl.ds(base, seq_len + ROWS_PER_BLOCK), :]
    for r in range(ROWS_PER_BLOCK):
        off = ROWS_PER_BLOCK - r
        out_ref[r] = xv + sup[off : off + seq_len]


def _tc_part(x, embed_table):
    batch, seq_len, d_model = x.shape
    grid = (TC_ROWS // ROWS_PER_BLOCK,)
    return pl.pallas_call(
        _rpe_block,
        grid=grid,
        in_specs=[
            pl.BlockSpec((batch, seq_len, d_model), lambda i: (0, 0, 0)),
            pl.BlockSpec(embed_table.shape, lambda i: (0, 0)),
        ],
        out_specs=pl.BlockSpec(
            (ROWS_PER_BLOCK, seq_len, d_model), lambda i: (i, 0, 0)
        ),
        out_shape=jax.ShapeDtypeStruct((TC_ROWS, seq_len, d_model), x.dtype),
        compiler_params=pltpu.CompilerParams(
            dimension_semantics=("parallel",)
        ),
    )(x, embed_table)


def _sc_part(x, embed_table):
    batch, seq_len, d_model = x.shape
    n_tbl = embed_table.shape[0]
    n_workers = NUM_CORES * NUM_SUBCORES
    sc_rows = seq_len - TC_ROWS
    rows_per_w = sc_rows // n_workers
    row_elems = seq_len * d_model
    n_chunks = row_elems // CHUNK
    sup_elems = CHUNK + (rows_per_w - 1) * d_model

    x_flat = x.reshape(row_elems)
    tbl_flat = embed_table.reshape(n_tbl * d_model)

    mesh = plsc.VectorSubcoreMesh(
        core_axis_name="c", subcore_axis_name="s"
    )

    @functools.partial(
        pl.kernel,
        mesh=mesh,
        out_type=jax.ShapeDtypeStruct((sc_rows * row_elems,), jnp.float32),
        scratch_types=[
            pltpu.VMEM((CHUNK,), jnp.float32),
            pltpu.VMEM((sup_elems,), jnp.float32),
            [pltpu.VMEM((CHUNK,), jnp.float32) for _ in range(NBUF)],
            [pltpu.SemaphoreType.DMA for _ in range(NBUF)],
        ],
    )
    def rpe_sc(x_hbm, tbl_hbm, out_hbm, xb, sup, obufs, sems_out):
        wid = lax.axis_index("s") * NUM_CORES + lax.axis_index("c")
        i0 = TC_ROWS + wid * rows_per_w

        def chunk_body(c, _):
            j_off = c * CHUNK
            pltpu.sync_copy(x_hbm.at[pl.ds(j_off, CHUNK)], xb)
            s_off = (seq_len - (i0 + rows_per_w - 1)) * d_model + j_off
            pltpu.sync_copy(tbl_hbm.at[pl.ds(s_off, sup_elems)], sup)

            def out_copy(r):
                o_off = (wid * rows_per_w + r) * row_elems + j_off
                return pltpu.make_async_copy(
                    obufs[r % NBUF],
                    out_hbm.at[pl.ds(o_off, CHUNK)],
                    sems_out[r % NBUF],
                )

            for r in range(rows_per_w):
                b = r % NBUF
                if r >= NBUF:
                    out_copy(r - NBUF).wait()
                w_off = (rows_per_w - 1 - r) * d_model

                @plsc.parallel_loop(0, CHUNK, step=LANES, unroll=8)
                def add_body(o):
                    obufs[b][pl.ds(o, LANES)] = (
                        sup[pl.ds(o + w_off, LANES)]
                        + xb[pl.ds(o, LANES)]
                    )
                out_copy(r).start()
            for r in range(max(rows_per_w - NBUF, 0), rows_per_w):
                out_copy(r).wait()
            return 0

        lax.fori_loop(0, n_chunks, chunk_body, 0)

    out_flat = rpe_sc(x_flat, tbl_flat)
    return out_flat.reshape(sc_rows, seq_len, d_model)


def kernel(x, embed_table):
    top = _tc_part(x, embed_table)
    bot = _sc_part(x, embed_table)
    return jnp.concatenate([top, bot], axis=0)


# speedup vs baseline: 7.4606x; 4.1819x over previous
import functools

import jax
import jax.numpy as jnp
from jax.experimental import pallas as pl
from jax.experimental.pallas import tpu as pltpu


ROWS_PER_BLOCK = 16


def _rpe_block(x_ref, tbl_ref, out_ref):
    seq_len = x_ref.shape[1]
    i0 = pl.program_id(0) * ROWS_PER_BLOCK
    xv = x_ref[0]
    base = pl.multiple_of(seq_len - i0 - ROWS_PER_BLOCK, 8)
    sup = tbl_ref[pl.ds(base, seq_len + ROWS_PER_BLOCK), :]
    for r in range(ROWS_PER_BLOCK):
        off = ROWS_PER_BLOCK - r
        out_ref[r] = xv + sup[off : off + seq_len]


def kernel(x, embed_table):
    batch, seq_len, d_model = x.shape
    grid = (seq_len // ROWS_PER_BLOCK,)
    out = pl.pallas_call(
        _rpe_block,
        grid=grid,
        in_specs=[
            pl.BlockSpec((batch, seq_len, d_model), lambda i: (0, 0, 0)),
            pl.BlockSpec(embed_table.shape, lambda i: (0, 0)),
        ],
        out_specs=pl.BlockSpec(
            (ROWS_PER_BLOCK, seq_len, d_model), lambda i: (i, 0, 0)
        ),
        out_shape=jax.ShapeDtypeStruct((seq_len, seq_len, d_model), x.dtype),
        compiler_params=pltpu.CompilerParams(
            dimension_semantics=("parallel",)
        ),
    )(x, embed_table)
    return out
